# baseline (speedup 1.0000x reference)
import jax
import jax.numpy as jnp
from jax import lax
from jax.experimental import pallas as pl
from jax.experimental.pallas import tpu as pltpu
from jax.experimental.pallas import tpu_sc as plsc

_F = 26
_V = 100000
_D = 64
_B = 4096
_L = 16

_NC = 2
_NS = 16
_NW = _NC * _NS
_BPW = _B // _NW
_IPW = _BPW * _F
_RPC = 4
_IPC = _RPC * _F
_NCHUNK = _BPW // _RPC


def _tree_sum(vals):
    while len(vals) > 1:
        nxt = [vals[i] + vals[i + 1] for i in range(0, len(vals) - 1, 2)]
        if len(vals) % 2:
            nxt.append(vals[-1])
        vals = nxt
    return vals[0]


def _sc_body(x_hbm, tab_hbm, out_hbm, xv, idxv, bufv, outv, sem):
    cid = lax.axis_index("c")
    sid = lax.axis_index("s")
    wid = sid * _NC + cid

    ibase = pl.multiple_of(wid * _IPW, 8)
    pltpu.sync_copy(x_hbm.at[pl.ds(ibase, _IPW)], xv)

    def mkidx(j, carry):
        p = pl.multiple_of(j * _L, _L)
        lane = lax.iota(jnp.int32, _L) + p
        offs = lax.rem(lane, _F) * _V
        idxv[pl.ds(p, _L)] = xv[pl.ds(p, _L)] + offs
        return carry

    lax.fori_loop(0, _IPW // _L, mkidx, 0)

    def chunk(ci, carry):
        start = pl.multiple_of(ci * _IPC, 8)
        cp = pltpu.make_async_copy(
            tab_hbm.at[idxv.at[pl.ds(start, _IPC)]], bufv, sem)
        cp.start()
        cp.wait()
        obase = ci * (_RPC * _D)
        for r in range(_RPC):
            for dd in range(_D // _L):
                vals = [bufv[r * _F + f, pl.ds(dd * _L, _L)] for f in range(_F)]
                outv[pl.ds(obase + r * _D + dd * _L, _L)] = _tree_sum(vals)
        return carry

    lax.fori_loop(0, _NCHUNK, chunk, 0)

    obase = pl.multiple_of(wid * (_BPW * _D), 8)
    pltpu.sync_copy(outv, out_hbm.at[pl.ds(obase, _BPW * _D)])


def kernel(x, tables):
    xf = x.reshape(-1).astype(jnp.int32)
    tf = tables.reshape(_F * _V, _D)
    mesh = plsc.VectorSubcoreMesh(core_axis_name="c", subcore_axis_name="s")
    run = pl.kernel(
        _sc_body,
        mesh=mesh,
        compiler_params=pltpu.CompilerParams(use_tc_tiling_on_sc=False),
        out_type=jax.ShapeDtypeStruct((_B * _D,), jnp.float32),
        scratch_types=[
            pltpu.VMEM((_IPW,), jnp.int32),
            pltpu.VMEM((_IPW,), jnp.int32),
            pltpu.VMEM((_IPC, _D), jnp.float32),
            pltpu.VMEM((_BPW * _D,), jnp.float32),
            pltpu.SemaphoreType.DMA,
        ],
    )
    out = run(xf, tf)
    return out.reshape(_B, _D)


# speedup vs baseline: 3.7074x; 3.7074x over previous
import jax
import jax.numpy as jnp
from jax import lax
from jax.experimental import pallas as pl
from jax.experimental.pallas import tpu as pltpu
from jax.experimental.pallas import tpu_sc as plsc

_F = 26
_V = 100000
_D = 64
_B = 4096
_L = 16

_NC = 2
_NS = 16
_NW = _NC * _NS
_DPW = _D // _NW


def _sc_body(xt_hbm, tt_hbm, out_hbm, xbuf, rowbuf, acc):
    cid = lax.axis_index("c")
    sid = lax.axis_index("s")
    wid = sid * _NC + cid
    d0 = wid * _DPW

    zero = jnp.zeros((_L,), jnp.float32)

    def zacc(j, carry):
        p = pl.multiple_of(j * _L, _L)
        acc[0, pl.ds(p, _L)] = zero
        acc[1, pl.ds(p, _L)] = zero
        return carry

    lax.fori_loop(0, _B // _L, zacc, 0)

    def field(f, carry):
        pltpu.sync_copy(xt_hbm.at[f], xbuf)
        for dd in range(_DPW):
            row = f * _D + d0 + dd
            pltpu.sync_copy(tt_hbm.at[row], rowbuf)

            def bgroup(j, carry2):
                p = pl.multiple_of(j * _L, _L)
                idx = xbuf[pl.ds(p, _L)]
                vals = plsc.load_gather(rowbuf, [idx])
                acc[dd, pl.ds(p, _L)] += vals
                return carry2

            lax.fori_loop(0, _B // _L, bgroup, 0)
        return carry

    lax.fori_loop(0, _F, field, 0)

    for dd in range(_DPW):
        pltpu.sync_copy(acc.at[dd], out_hbm.at[d0 + dd])


def kernel(x, tables):
    xt = x.T.astype(jnp.int32)
    tt = tables.transpose(0, 2, 1).reshape(_F * _D, _V)
    mesh = plsc.VectorSubcoreMesh(core_axis_name="c", subcore_axis_name="s")
    run = pl.kernel(
        _sc_body,
        mesh=mesh,
        compiler_params=pltpu.CompilerParams(needs_layout_passes=False),
        out_type=jax.ShapeDtypeStruct((_D, _B), jnp.float32),
        scratch_types=[
            pltpu.VMEM((_B,), jnp.int32),
            pltpu.VMEM((_V,), jnp.float32),
            pltpu.VMEM((_DPW, _B), jnp.float32),
        ],
    )
    out = run(xt, tt)
    return out.T


# speedup vs baseline: 3.8835x; 1.0475x over previous
import jax
import jax.numpy as jnp
from jax import lax
from jax.experimental import pallas as pl
from jax.experimental.pallas import tpu as pltpu
from jax.experimental.pallas import tpu_sc as plsc

_F = 26
_V = 100000
_D = 64
_B = 4096
_L = 16

_UNROLL = 8

_NC = 2
_NS = 16
_NW = _NC * _NS
_DPW = _D // _NW


def _sc_body(xt_hbm, tt_hbm, out_hbm, xbuf, rowbuf, acc):
    cid = lax.axis_index("c")
    sid = lax.axis_index("s")
    wid = sid * _NC + cid
    d0 = wid * _DPW

    zero = jnp.zeros((_L,), jnp.float32)

    def zacc(j, carry):
        p = pl.multiple_of(j * _L, _L)
        acc[0, pl.ds(p, _L)] = zero
        acc[1, pl.ds(p, _L)] = zero
        return carry

    lax.fori_loop(0, _B // _L, zacc, 0)

    def field(f, carry):
        pltpu.sync_copy(xt_hbm.at[f], xbuf)
        for dd in range(_DPW):
            row = f * _D + d0 + dd
            pltpu.sync_copy(tt_hbm.at[row], rowbuf)

            def bgroup(j, carry2):
                base = pl.multiple_of(j * (_L * _UNROLL), _L)
                for u in range(_UNROLL):
                    p = base + u * _L
                    idx = xbuf[pl.ds(p, _L)]
                    vals = plsc.load_gather(rowbuf, [idx])
                    plsc.addupdate(acc.at[dd, pl.ds(p, _L)], vals)
                return carry2

            lax.fori_loop(0, _B // (_L * _UNROLL), bgroup, 0)
        return carry

    lax.fori_loop(0, _F, field, 0)

    for dd in range(_DPW):
        pltpu.sync_copy(acc.at[dd], out_hbm.at[d0 + dd])


def kernel(x, tables):
    xt = x.T.astype(jnp.int32)
    tt = tables.transpose(0, 2, 1).reshape(_F * _D, _V)
    mesh = plsc.VectorSubcoreMesh(core_axis_name="c", subcore_axis_name="s")
    run = pl.kernel(
        _sc_body,
        mesh=mesh,
        compiler_params=pltpu.CompilerParams(needs_layout_passes=False),
        out_type=jax.ShapeDtypeStruct((_D, _B), jnp.float32),
        scratch_types=[
            pltpu.VMEM((_B,), jnp.int32),
            pltpu.VMEM((_V,), jnp.float32),
            pltpu.VMEM((_DPW, _B), jnp.float32),
        ],
    )
    out = run(xt, tt)
    return out.T
